# baseline (speedup 1.0000x reference)
import functools

import jax
import jax.numpy as jnp
from jax import lax
from jax.experimental import pallas as pl
from jax.experimental.pallas import tpu as pltpu
from jax.experimental.pallas import tpu_sc as plsc

N = 10000
D = 256
HD = 128
E = 160000
M = 10240
E_PAD = 163840
EPT = E_PAD // 16
K = 80
NB = 4
NCH = EPT // K
RPT = M // 16
KD = 128


@functools.cache
def _mesh():
    return plsc.VectorSubcoreMesh(
        core_axis_name="c", subcore_axis_name="s", num_cores=2, num_subcores=16
    )


def _deg_body(dst_hbm, deg_hbm, deg_sh, ones_v, idx_v, zero_v, semi0, semi1):
    c = lax.axis_index("c")
    s = lax.axis_index("s")
    for k in range(KD // 16):
        ones_v[pl.ds(k * 16, 16)] = jnp.ones((16,), jnp.float32)
    for k in range(RPT // 16):
        zero_v[pl.ds(k * 16, 16)] = jnp.zeros((16,), jnp.float32)
    pltpu.sync_copy(zero_v, deg_sh.at[pl.ds(s * RPT, RPT)])
    plsc.subcore_barrier()

    base = (s * 2 + c) * (E_PAD // 32)
    nld = E_PAD // 32 // (4 * KD)
    pltpu.async_copy(dst_hbm.at[pl.ds(base, 4 * KD)], idx_v.at[0], semi0)
    pltpu.async_copy(dst_hbm.at[pl.ds(base + 4 * KD, 4 * KD)],
                     idx_v.at[1], semi1)

    def body(jj, carry):
        for b in (0, 1):
            ld = jj * 2 + b
            sem = semi0 if b == 0 else semi1
            pltpu.make_async_copy(
                dst_hbm.at[pl.ds(base, 4 * KD)], idx_v.at[b], sem
            ).wait()
            for q in range(4):
                pltpu.sync_copy(
                    ones_v, deg_sh.at[idx_v.at[b, pl.ds(q * KD, KD)]], add=True
                )

            @pl.when(ld + 2 < nld)
            def _():
                pltpu.async_copy(
                    dst_hbm.at[pl.ds(base + (ld + 2) * 4 * KD, 4 * KD)],
                    idx_v.at[b], sem,
                )
        return carry

    lax.fori_loop(0, nld // 2, body, 0)
    plsc.subcore_barrier()
    pltpu.sync_copy(
        deg_sh.at[pl.ds(s * RPT, RPT)],
        deg_hbm.at[c, pl.ds(s * RPT, RPT)],
    )


@functools.cache
def _deg_call():
    return pl.kernel(
        _deg_body,
        out_type=jax.ShapeDtypeStruct((2, M), jnp.float32),
        mesh=_mesh(),
        scratch_types=[
            pltpu.VMEM_SHARED((M,), jnp.float32),
            pltpu.VMEM((KD,), jnp.float32),
            pltpu.VMEM((2, 4 * KD), jnp.int32),
            pltpu.VMEM((RPT,), jnp.float32),
            pltpu.SemaphoreType.DMA,
            pltpu.SemaphoreType.DMA,
        ],
        compiler_params=pltpu.CompilerParams(use_tc_tiling_on_sc=False),
    )


NI = 8
PD = 6


def _segsum_body(y_hbm, pk_hbm, out_hbm,
                 acc_sh, idxbuf, rowbuf,
                 g0, g1, g2, g3, t0, t1, t2, t3,
                 i0, i1, i2, i3, i4, i5, i6, i7):
    c = lax.axis_index("c")
    s = lax.axis_index("s")
    base = s * EPT
    gsem = (g0, g1, g2, g3)
    ssem = (t0, t1, t2, t3)
    isem = (i0, i1, i2, i3, i4, i5, i6, i7)

    def idx_fire(ch, q):
        pltpu.async_copy(
            pk_hbm.at[c, :, pl.ds(base + ch * K, K)], idxbuf.at[q], isem[q]
        )

    def idx_wait(q):
        pltpu.make_async_copy(
            pk_hbm.at[c, :, pl.ds(base, K)], idxbuf.at[q], isem[q]
        ).wait()

    def gat_fire(b, q):
        pltpu.async_copy(y_hbm.at[idxbuf.at[q, 0]], rowbuf.at[b], gsem[b])

    def gat_wait(b, q):
        pltpu.make_async_copy(
            y_hbm.at[idxbuf.at[q, 0]], rowbuf.at[b], gsem[b]
        ).wait()

    def sct_fire(b, q):
        pltpu.async_copy(
            rowbuf.at[b], acc_sh.at[idxbuf.at[q, 1]], ssem[b], add=True
        )

    def sct_wait(b, q):
        pltpu.make_async_copy(
            rowbuf.at[b], acc_sh.at[idxbuf.at[q, 1]], ssem[b]
        ).wait()

    for q in range(PD):
        idx_fire(q, q)

    def zbody(i, carry):
        for k in range(HD // 16):
            rowbuf[0, i, pl.ds(k * 16, 16)] = jnp.zeros((16,), jnp.float32)
        return carry

    lax.fori_loop(0, K, zbody, 0)
    r0 = s * RPT
    for t in range(RPT // K):
        pltpu.sync_copy(rowbuf.at[0], acc_sh.at[pl.ds(r0 + t * K, K)])
    plsc.subcore_barrier()

    idx_wait(0)
    gat_fire(0, 0)
    idx_wait(1)
    gat_fire(1, 1)

    def mbody(jj, carry):
        for q in range(NI):
            ch = jj * NI + q
            b = q % NB
            gat_wait(b, q)
            sct_fire(b, q)
            b2 = (b + 2) % NB
            q2 = (q + 2) % NI

            @pl.when(ch + 2 < NCH)
            def _():
                @pl.when(ch >= 2)
                def _():
                    sct_wait(b2, q2)
                idx_wait(q2)
                gat_fire(b2, q2)

                @pl.when(ch + PD < NCH)
                def _():
                    idx_fire(ch + PD, (q + PD) % NI)
        return carry

    lax.fori_loop(0, NCH // NI, mbody, 0)
    for b in range(NB):
        sct_wait(b, b)
    plsc.subcore_barrier()

    pltpu.sync_copy(acc_sh.at[pl.ds(r0, RPT)],
                    out_hbm.at[c, pl.ds(r0, RPT)])


@functools.cache
def _segsum_call():
    return pl.kernel(
        _segsum_body,
        out_type=jax.ShapeDtypeStruct((2, M, HD), jnp.float32),
        mesh=_mesh(),
        scratch_types=[
            pltpu.VMEM_SHARED((M, HD), jnp.float32),
            pltpu.VMEM((NI, 2, K), jnp.int32),
            pltpu.VMEM((NB, K, HD), jnp.float32),
        ] + [pltpu.SemaphoreType.DMA] * 16,
        compiler_params=pltpu.CompilerParams(use_tc_tiling_on_sc=False),
    )


def _mm_scale_k(x_ref, w_ref, dinv_ref, y_ref):
    xw = jnp.dot(x_ref[...], w_ref[...], preferred_element_type=jnp.float32)
    xw = xw * dinv_ref[...]
    y_ref[0] = xw[:, :HD]
    y_ref[1] = xw[:, HD:]


def _mid_k(s_ref, y_ref, dinv_ref, b_ref, w_ref, o_ref):
    h0 = jnp.maximum(
        dinv_ref[...] * (s_ref[0] + y_ref[0]) + b_ref[:, :HD], 0.0
    )
    h1 = jnp.maximum(
        dinv_ref[...] * (s_ref[1] + y_ref[1]) + b_ref[:, HD:], 0.0
    )
    o = (
        jnp.dot(h0, w_ref[:HD, :], preferred_element_type=jnp.float32)
        + jnp.dot(h1, w_ref[HD:, :], preferred_element_type=jnp.float32)
    ) * dinv_ref[...]
    o_ref[0] = o[:, :HD]
    o_ref[1] = o[:, HD:]


def _fin_k(s_ref, y_ref, dinv_ref, b_ref, o_ref):
    o_ref[:, :HD] = jnp.maximum(
        dinv_ref[...] * (s_ref[0] + y_ref[0]) + b_ref[:, :HD], 0.0
    )
    o_ref[:, HD:] = jnp.maximum(
        dinv_ref[...] * (s_ref[1] + y_ref[1]) + b_ref[:, HD:], 0.0
    )


def _mm_scale(x, w, dinv_col, bm=512):
    g = M // bm
    return pl.pallas_call(
        _mm_scale_k,
        grid=(g,),
        in_specs=[
            pl.BlockSpec((bm, D), lambda i: (i, 0)),
            pl.BlockSpec((D, D), lambda i: (0, 0)),
            pl.BlockSpec((bm, 1), lambda i: (i, 0)),
        ],
        out_specs=pl.BlockSpec((2, bm, HD), lambda i: (0, i, 0)),
        out_shape=jax.ShapeDtypeStruct((2, M, HD), jnp.float32),
    )(x, w, dinv_col)


def _mid(s1, y1, dinv_col, b1, w2, bm=512):
    g = M // bm
    return pl.pallas_call(
        _mid_k,
        grid=(g,),
        in_specs=[
            pl.BlockSpec((2, bm, HD), lambda i: (0, i, 0)),
            pl.BlockSpec((2, bm, HD), lambda i: (0, i, 0)),
            pl.BlockSpec((bm, 1), lambda i: (i, 0)),
            pl.BlockSpec((1, D), lambda i: (0, 0)),
            pl.BlockSpec((D, D), lambda i: (0, 0)),
        ],
        out_specs=pl.BlockSpec((2, bm, HD), lambda i: (0, i, 0)),
        out_shape=jax.ShapeDtypeStruct((2, M, HD), jnp.float32),
    )(s1, y1, dinv_col, b1, w2)


def _fin(s2, y2, dinv_col, b2, bm=1000):
    g = N // bm
    return pl.pallas_call(
        _fin_k,
        grid=(g,),
        in_specs=[
            pl.BlockSpec((2, bm, HD), lambda i: (0, i, 0)),
            pl.BlockSpec((2, bm, HD), lambda i: (0, i, 0)),
            pl.BlockSpec((bm, 1), lambda i: (i, 0)),
            pl.BlockSpec((1, D), lambda i: (0, 0)),
        ],
        out_specs=pl.BlockSpec((bm, D), lambda i: (i, 0)),
        out_shape=jax.ShapeDtypeStruct((N, D), jnp.float32),
    )(s2, y2, dinv_col, b2)


@jax.jit
def kernel(x, edge_index, W1, b1, W2, b2):
    src = edge_index[0]
    dst = edge_index[1]
    pad = jnp.arange(E_PAD - E, dtype=jnp.int32)
    src_p = jnp.concatenate([src, (pad * 131) % N])
    dst_p = jnp.concatenate([dst, N + (pad & 127)])
    pk = jnp.stack([
        jnp.stack([src_p, dst_p]),
        jnp.stack([src_p + M, dst_p]),
    ])

    x_pad = jnp.pad(x, ((0, M - N), (0, 0)))

    deg = _deg_call()(dst_p)
    dinv_col = lax.rsqrt(deg[0] + deg[1] + 1.0)[:, None]

    y1 = _mm_scale(x_pad, W1, dinv_col)
    s1 = _segsum_call()(y1.reshape(2 * M, HD), pk)
    y2 = _mid(s1, y1, dinv_col, b1[None, :], W2)
    s2 = _segsum_call()(y2.reshape(2 * M, HD), pk)
    return _fin(s2, y2, dinv_col, b2[None, :])


# speedup vs baseline: 1.1103x; 1.1103x over previous
import functools

import jax
import jax.numpy as jnp
from jax import lax
from jax.experimental import pallas as pl
from jax.experimental.pallas import tpu as pltpu
from jax.experimental.pallas import tpu_sc as plsc

N = 10000
D = 256
HD = 128
E = 160000
M = 10240
E_PAD = 163840
EPT = E_PAD // 16
K = 128
NCH = EPT // K
RPT = M // 16
KD = 128


@functools.cache
def _mesh():
    return plsc.VectorSubcoreMesh(
        core_axis_name="c", subcore_axis_name="s", num_cores=2, num_subcores=16
    )


def _deg_body(dst_hbm, deg_hbm, deg_sh, ones_v, idx_v, zero_v, semi0, semi1):
    c = lax.axis_index("c")
    s = lax.axis_index("s")
    for k in range(KD // 16):
        ones_v[pl.ds(k * 16, 16)] = jnp.ones((16,), jnp.float32)
    for k in range(RPT // 16):
        zero_v[pl.ds(k * 16, 16)] = jnp.zeros((16,), jnp.float32)
    pltpu.sync_copy(zero_v, deg_sh.at[pl.ds(s * RPT, RPT)])
    plsc.subcore_barrier()

    base = (s * 2 + c) * (E_PAD // 32)
    nld = E_PAD // 32 // (4 * KD)
    pltpu.async_copy(dst_hbm.at[pl.ds(base, 4 * KD)], idx_v.at[0], semi0)
    pltpu.async_copy(dst_hbm.at[pl.ds(base + 4 * KD, 4 * KD)],
                     idx_v.at[1], semi1)

    def body(jj, carry):
        for b in (0, 1):
            ld = jj * 2 + b
            sem = semi0 if b == 0 else semi1
            pltpu.make_async_copy(
                dst_hbm.at[pl.ds(base, 4 * KD)], idx_v.at[b], sem
            ).wait()
            for q in range(4):
                pltpu.sync_copy(
                    ones_v, deg_sh.at[idx_v.at[b, pl.ds(q * KD, KD)]], add=True
                )

            @pl.when(ld + 2 < nld)
            def _():
                pltpu.async_copy(
                    dst_hbm.at[pl.ds(base + (ld + 2) * 4 * KD, 4 * KD)],
                    idx_v.at[b], sem,
                )
        return carry

    lax.fori_loop(0, nld // 2, body, 0)
    plsc.subcore_barrier()
    pltpu.sync_copy(
        deg_sh.at[pl.ds(s * RPT, RPT)],
        deg_hbm.at[c, pl.ds(s * RPT, RPT)],
    )


@functools.cache
def _deg_call():
    return pl.kernel(
        _deg_body,
        out_type=jax.ShapeDtypeStruct((2, M), jnp.float32),
        mesh=_mesh(),
        scratch_types=[
            pltpu.VMEM_SHARED((M,), jnp.float32),
            pltpu.VMEM((KD,), jnp.float32),
            pltpu.VMEM((2, 4 * KD), jnp.int32),
            pltpu.VMEM((RPT,), jnp.float32),
            pltpu.SemaphoreType.DMA,
            pltpu.SemaphoreType.DMA,
        ],
        compiler_params=pltpu.CompilerParams(use_tc_tiling_on_sc=False),
    )


def _segsum_body(y_hbm, src2_hbm, dst_hbm, out_hbm,
                 acc_sh, src_in, dstbuf, rowbuf,
                 sem0, sem1, semd0, semd1, sems):
    c = lax.axis_index("c")
    s = lax.axis_index("s")
    base = s * EPT

    pltpu.async_copy(src2_hbm.at[c, pl.ds(base, EPT)], src_in, sems)

    def zbody(i, carry):
        for k in range(HD // 16):
            rowbuf[0, i, pl.ds(k * 16, 16)] = jnp.zeros((16,), jnp.float32)
        return carry

    lax.fori_loop(0, K, zbody, 0)
    r0 = s * RPT
    for t in range(RPT // K):
        pltpu.sync_copy(rowbuf.at[0], acc_sh.at[pl.ds(r0 + t * K, K)])
    pltpu.make_async_copy(src2_hbm.at[c, pl.ds(base, EPT)], src_in, sems).wait()
    plsc.subcore_barrier()

    pltpu.async_copy(y_hbm.at[src_in.at[pl.ds(0, K)]], rowbuf.at[0], sem0)
    pltpu.async_copy(dst_hbm.at[pl.ds(base, K)], dstbuf.at[0], semd0)
    pltpu.async_copy(y_hbm.at[src_in.at[pl.ds(K, K)]], rowbuf.at[1], sem1)
    pltpu.async_copy(dst_hbm.at[pl.ds(base + K, K)], dstbuf.at[1], semd1)

    def mbody(jj, carry):
        for b in (0, 1):
            ch = jj * 2 + b
            sem = sem0 if b == 0 else sem1
            semd = semd0 if b == 0 else semd1
            pltpu.make_async_copy(
                y_hbm.at[src_in.at[pl.ds(0, K)]], rowbuf.at[b], sem
            ).wait()
            pltpu.make_async_copy(
                dst_hbm.at[pl.ds(base, K)], dstbuf.at[b], semd
            ).wait()
            pltpu.sync_copy(
                rowbuf.at[b], acc_sh.at[dstbuf.at[b]], add=True
            )

            @pl.when(ch + 2 < NCH)
            def _():
                pltpu.async_copy(
                    y_hbm.at[src_in.at[pl.ds((ch + 2) * K, K)]],
                    rowbuf.at[b], sem,
                )
                pltpu.async_copy(
                    dst_hbm.at[pl.ds(base + (ch + 2) * K, K)],
                    dstbuf.at[b], semd,
                )
        return carry

    lax.fori_loop(0, NCH // 2, mbody, 0)
    plsc.subcore_barrier()

    pltpu.sync_copy(acc_sh.at[pl.ds(r0, RPT)],
                    out_hbm.at[c, pl.ds(r0, RPT)])


@functools.cache
def _segsum_call():
    return pl.kernel(
        _segsum_body,
        out_type=jax.ShapeDtypeStruct((2, M, HD), jnp.float32),
        mesh=_mesh(),
        scratch_types=[
            pltpu.VMEM_SHARED((M, HD), jnp.float32),
            pltpu.VMEM((EPT,), jnp.int32),
            pltpu.VMEM((2, K), jnp.int32),
            pltpu.VMEM((2, K, HD), jnp.float32),
        ] + [pltpu.SemaphoreType.DMA] * 5,
        compiler_params=pltpu.CompilerParams(use_tc_tiling_on_sc=False),
    )


def _mm_scale_k(x_ref, w_ref, dinv_ref, y_ref):
    xw = jnp.dot(x_ref[...], w_ref[...], preferred_element_type=jnp.float32)
    xw = xw * dinv_ref[...]
    y_ref[0] = xw[:, :HD]
    y_ref[1] = xw[:, HD:]


def _mid_k(s_ref, y_ref, dinv_ref, b_ref, w_ref, o_ref):
    h0 = jnp.maximum(
        dinv_ref[...] * (s_ref[0] + y_ref[0]) + b_ref[:, :HD], 0.0
    )
    h1 = jnp.maximum(
        dinv_ref[...] * (s_ref[1] + y_ref[1]) + b_ref[:, HD:], 0.0
    )
    o = (
        jnp.dot(h0, w_ref[:HD, :], preferred_element_type=jnp.float32)
        + jnp.dot(h1, w_ref[HD:, :], preferred_element_type=jnp.float32)
    ) * dinv_ref[...]
    o_ref[0] = o[:, :HD]
    o_ref[1] = o[:, HD:]


def _fin_k(s_ref, y_ref, dinv_ref, b_ref, o_ref):
    o_ref[:, :HD] = jnp.maximum(
        dinv_ref[...] * (s_ref[0] + y_ref[0]) + b_ref[:, :HD], 0.0
    )
    o_ref[:, HD:] = jnp.maximum(
        dinv_ref[...] * (s_ref[1] + y_ref[1]) + b_ref[:, HD:], 0.0
    )


def _mm_scale(x, w, dinv_col, bm=1024):
    g = M // bm
    return pl.pallas_call(
        _mm_scale_k,
        grid=(g,),
        in_specs=[
            pl.BlockSpec((bm, D), lambda i: (i, 0)),
            pl.BlockSpec((D, D), lambda i: (0, 0)),
            pl.BlockSpec((bm, 1), lambda i: (i, 0)),
        ],
        out_specs=pl.BlockSpec((2, bm, HD), lambda i: (0, i, 0)),
        out_shape=jax.ShapeDtypeStruct((2, M, HD), jnp.float32),
    )(x, w, dinv_col)


def _mid(s1, y1, dinv_col, b1, w2, bm=1024):
    g = M // bm
    return pl.pallas_call(
        _mid_k,
        grid=(g,),
        in_specs=[
            pl.BlockSpec((2, bm, HD), lambda i: (0, i, 0)),
            pl.BlockSpec((2, bm, HD), lambda i: (0, i, 0)),
            pl.BlockSpec((bm, 1), lambda i: (i, 0)),
            pl.BlockSpec((1, D), lambda i: (0, 0)),
            pl.BlockSpec((D, D), lambda i: (0, 0)),
        ],
        out_specs=pl.BlockSpec((2, bm, HD), lambda i: (0, i, 0)),
        out_shape=jax.ShapeDtypeStruct((2, M, HD), jnp.float32),
    )(s1, y1, dinv_col, b1, w2)


def _fin(s2, y2, dinv_col, b2, bm=2000):
    g = N // bm
    return pl.pallas_call(
        _fin_k,
        grid=(g,),
        in_specs=[
            pl.BlockSpec((2, bm, HD), lambda i: (0, i, 0)),
            pl.BlockSpec((2, bm, HD), lambda i: (0, i, 0)),
            pl.BlockSpec((bm, 1), lambda i: (i, 0)),
            pl.BlockSpec((1, D), lambda i: (0, 0)),
        ],
        out_specs=pl.BlockSpec((bm, D), lambda i: (i, 0)),
        out_shape=jax.ShapeDtypeStruct((N, D), jnp.float32),
    )(s2, y2, dinv_col, b2)


@jax.jit
def kernel(x, edge_index, W1, b1, W2, b2):
    src = edge_index[0]
    dst = edge_index[1]
    pad = jnp.arange(E_PAD - E, dtype=jnp.int32)
    src_p = jnp.concatenate([src, (pad * 131) % N])
    dst_p = jnp.concatenate([dst, N + (pad & 127)])
    src2 = jnp.stack([src_p, src_p + M])

    x_pad = jnp.pad(x, ((0, M - N), (0, 0)))

    deg = _deg_call()(dst_p)
    dinv_col = lax.rsqrt(deg[0] + deg[1] + 1.0)[:, None]

    y1 = _mm_scale(x_pad, W1, dinv_col)
    s1 = _segsum_call()(y1.reshape(2 * M, HD), src2, dst_p)
    y2 = _mid(s1, y1, dinv_col, b1[None, :], W2)
    s2 = _segsum_call()(y2.reshape(2 * M, HD), src2, dst_p)
    return _fin(s2, y2, dinv_col, b2[None, :])


# speedup vs baseline: 1.1138x; 1.0032x over previous
import functools

import jax
import jax.numpy as jnp
from jax import lax
from jax.experimental import pallas as pl
from jax.experimental.pallas import tpu as pltpu
from jax.experimental.pallas import tpu_sc as plsc

N = 10000
D = 256
HD = 128
E = 160000
M = 10240
E_PAD = 163840
EPT = E_PAD // 16
K = 128
NCH = EPT // K
RPT = M // 16
KD = 128


@functools.cache
def _mesh():
    return plsc.VectorSubcoreMesh(
        core_axis_name="c", subcore_axis_name="s", num_cores=2, num_subcores=16
    )


def _deg_body(dst_hbm, deg_hbm, deg_sh, ones_v, idx_v, zero_v, semi0, semi1):
    c = lax.axis_index("c")
    s = lax.axis_index("s")
    for k in range(KD // 16):
        ones_v[pl.ds(k * 16, 16)] = jnp.ones((16,), jnp.float32)
    for k in range(RPT // 16):
        zero_v[pl.ds(k * 16, 16)] = jnp.zeros((16,), jnp.float32)
    pltpu.sync_copy(zero_v, deg_sh.at[pl.ds(s * RPT, RPT)])
    plsc.subcore_barrier()

    base = (s * 2 + c) * (E_PAD // 32)
    nld = E_PAD // 32 // (4 * KD)
    pltpu.async_copy(dst_hbm.at[pl.ds(base, 4 * KD)], idx_v.at[0], semi0)
    pltpu.async_copy(dst_hbm.at[pl.ds(base + 4 * KD, 4 * KD)],
                     idx_v.at[1], semi1)

    def body(jj, carry):
        for b in (0, 1):
            ld = jj * 2 + b
            sem = semi0 if b == 0 else semi1
            pltpu.make_async_copy(
                dst_hbm.at[pl.ds(base, 4 * KD)], idx_v.at[b], sem
            ).wait()
            for q in range(4):
                pltpu.sync_copy(
                    ones_v, deg_sh.at[idx_v.at[b, pl.ds(q * KD, KD)]], add=True
                )

            @pl.when(ld + 2 < nld)
            def _():
                pltpu.async_copy(
                    dst_hbm.at[pl.ds(base + (ld + 2) * 4 * KD, 4 * KD)],
                    idx_v.at[b], sem,
                )
        return carry

    lax.fori_loop(0, nld // 2, body, 0)
    plsc.subcore_barrier()
    pltpu.sync_copy(
        deg_sh.at[pl.ds(s * RPT, RPT)],
        deg_hbm.at[c, pl.ds(s * RPT, RPT)],
    )


@functools.cache
def _deg_call():
    return pl.kernel(
        _deg_body,
        out_type=jax.ShapeDtypeStruct((2, M), jnp.float32),
        mesh=_mesh(),
        scratch_types=[
            pltpu.VMEM_SHARED((M,), jnp.float32),
            pltpu.VMEM((KD,), jnp.float32),
            pltpu.VMEM((2, 4 * KD), jnp.int32),
            pltpu.VMEM((RPT,), jnp.float32),
            pltpu.SemaphoreType.DMA,
            pltpu.SemaphoreType.DMA,
        ],
        compiler_params=pltpu.CompilerParams(use_tc_tiling_on_sc=False),
    )


def _segsum_body(y_hbm, src2_hbm, dst_hbm, out_hbm,
                 acc_sh, src_in, dstbuf, rowbuf,
                 sem0, sem1, semd0, semd1, sems, sca, scb):
    c = lax.axis_index("c")
    s = lax.axis_index("s")
    base = s * EPT

    pltpu.async_copy(src2_hbm.at[c, pl.ds(base, EPT)], src_in, sems)

    def zbody(i, carry):
        for k in range(HD // 16):
            rowbuf[0, i, pl.ds(k * 16, 16)] = jnp.zeros((16,), jnp.float32)
        return carry

    lax.fori_loop(0, K, zbody, 0)
    r0 = s * RPT
    for t in range(RPT // K):
        pltpu.sync_copy(rowbuf.at[0], acc_sh.at[pl.ds(r0 + t * K, K)])
    pltpu.make_async_copy(src2_hbm.at[c, pl.ds(base, EPT)], src_in, sems).wait()
    plsc.subcore_barrier()

    pltpu.async_copy(y_hbm.at[src_in.at[pl.ds(0, K)]], rowbuf.at[0], sem0)
    pltpu.async_copy(dst_hbm.at[pl.ds(base, K)], dstbuf.at[0], semd0)
    pltpu.async_copy(y_hbm.at[src_in.at[pl.ds(K, K)]], rowbuf.at[1], sem1)
    pltpu.async_copy(dst_hbm.at[pl.ds(base + K, K)], dstbuf.at[1], semd1)

    def mbody(jj, carry):
        for b in (0, 1):
            ch = jj * 2 + b
            sem = sem0 if b == 0 else sem1
            semd = semd0 if b == 0 else semd1
            pltpu.make_async_copy(
                y_hbm.at[src_in.at[pl.ds(0, K)]], rowbuf.at[b], sem
            ).wait()
            pltpu.make_async_copy(
                dst_hbm.at[pl.ds(base, K)], dstbuf.at[b], semd
            ).wait()
            pltpu.async_copy(
                rowbuf.at[b, pl.ds(0, K // 2)],
                acc_sh.at[dstbuf.at[b, pl.ds(0, K // 2)]], sca, add=True,
            )
            pltpu.async_copy(
                rowbuf.at[b, pl.ds(K // 2, K // 2)],
                acc_sh.at[dstbuf.at[b, pl.ds(K // 2, K // 2)]], scb, add=True,
            )
            pltpu.make_async_copy(
                rowbuf.at[b, pl.ds(0, K // 2)],
                acc_sh.at[dstbuf.at[b, pl.ds(0, K // 2)]], sca,
            ).wait()
            pltpu.make_async_copy(
                rowbuf.at[b, pl.ds(K // 2, K // 2)],
                acc_sh.at[dstbuf.at[b, pl.ds(K // 2, K // 2)]], scb,
            ).wait()

            @pl.when(ch + 2 < NCH)
            def _():
                pltpu.async_copy(
                    y_hbm.at[src_in.at[pl.ds((ch + 2) * K, K)]],
                    rowbuf.at[b], sem,
                )
                pltpu.async_copy(
                    dst_hbm.at[pl.ds(base + (ch + 2) * K, K)],
                    dstbuf.at[b], semd,
                )
        return carry

    lax.fori_loop(0, NCH // 2, mbody, 0)
    plsc.subcore_barrier()

    pltpu.sync_copy(acc_sh.at[pl.ds(r0, RPT)],
                    out_hbm.at[c, pl.ds(r0, RPT)])


@functools.cache
def _segsum_call():
    return pl.kernel(
        _segsum_body,
        out_type=jax.ShapeDtypeStruct((2, M, HD), jnp.float32),
        mesh=_mesh(),
        scratch_types=[
            pltpu.VMEM_SHARED((M, HD), jnp.float32),
            pltpu.VMEM((EPT,), jnp.int32),
            pltpu.VMEM((2, K), jnp.int32),
            pltpu.VMEM((2, K, HD), jnp.float32),
        ] + [pltpu.SemaphoreType.DMA] * 7,
        compiler_params=pltpu.CompilerParams(use_tc_tiling_on_sc=False),
    )


def _mm_scale_k(x_ref, w_ref, dinv_ref, y_ref):
    xw = jnp.dot(x_ref[...], w_ref[...], preferred_element_type=jnp.float32)
    xw = xw * dinv_ref[...]
    y_ref[0] = xw[:, :HD]
    y_ref[1] = xw[:, HD:]


def _mid_k(s_ref, y_ref, dinv_ref, b_ref, w_ref, o_ref):
    h0 = jnp.maximum(
        dinv_ref[...] * (s_ref[0] + y_ref[0]) + b_ref[:, :HD], 0.0
    )
    h1 = jnp.maximum(
        dinv_ref[...] * (s_ref[1] + y_ref[1]) + b_ref[:, HD:], 0.0
    )
    o = (
        jnp.dot(h0, w_ref[:HD, :], preferred_element_type=jnp.float32)
        + jnp.dot(h1, w_ref[HD:, :], preferred_element_type=jnp.float32)
    ) * dinv_ref[...]
    o_ref[0] = o[:, :HD]
    o_ref[1] = o[:, HD:]


def _fin_k(s_ref, y_ref, dinv_ref, b_ref, o_ref):
    o_ref[:, :HD] = jnp.maximum(
        dinv_ref[...] * (s_ref[0] + y_ref[0]) + b_ref[:, :HD], 0.0
    )
    o_ref[:, HD:] = jnp.maximum(
        dinv_ref[...] * (s_ref[1] + y_ref[1]) + b_ref[:, HD:], 0.0
    )


def _mm_scale(x, w, dinv_col, bm=1024):
    g = M // bm
    return pl.pallas_call(
        _mm_scale_k,
        grid=(g,),
        in_specs=[
            pl.BlockSpec((bm, D), lambda i: (i, 0)),
            pl.BlockSpec((D, D), lambda i: (0, 0)),
            pl.BlockSpec((bm, 1), lambda i: (i, 0)),
        ],
        out_specs=pl.BlockSpec((2, bm, HD), lambda i: (0, i, 0)),
        out_shape=jax.ShapeDtypeStruct((2, M, HD), jnp.float32),
    )(x, w, dinv_col)


def _mid(s1, y1, dinv_col, b1, w2, bm=1024):
    g = M // bm
    return pl.pallas_call(
        _mid_k,
        grid=(g,),
        in_specs=[
            pl.BlockSpec((2, bm, HD), lambda i: (0, i, 0)),
            pl.BlockSpec((2, bm, HD), lambda i: (0, i, 0)),
            pl.BlockSpec((bm, 1), lambda i: (i, 0)),
            pl.BlockSpec((1, D), lambda i: (0, 0)),
            pl.BlockSpec((D, D), lambda i: (0, 0)),
        ],
        out_specs=pl.BlockSpec((2, bm, HD), lambda i: (0, i, 0)),
        out_shape=jax.ShapeDtypeStruct((2, M, HD), jnp.float32),
    )(s1, y1, dinv_col, b1, w2)


def _fin(s2, y2, dinv_col, b2, bm=2000):
    g = N // bm
    return pl.pallas_call(
        _fin_k,
        grid=(g,),
        in_specs=[
            pl.BlockSpec((2, bm, HD), lambda i: (0, i, 0)),
            pl.BlockSpec((2, bm, HD), lambda i: (0, i, 0)),
            pl.BlockSpec((bm, 1), lambda i: (i, 0)),
            pl.BlockSpec((1, D), lambda i: (0, 0)),
        ],
        out_specs=pl.BlockSpec((bm, D), lambda i: (i, 0)),
        out_shape=jax.ShapeDtypeStruct((N, D), jnp.float32),
    )(s2, y2, dinv_col, b2)


@jax.jit
def kernel(x, edge_index, W1, b1, W2, b2):
    src = edge_index[0]
    dst = edge_index[1]
    pad = jnp.arange(E_PAD - E, dtype=jnp.int32)
    src_p = jnp.concatenate([src, (pad * 131) % N])
    dst_p = jnp.concatenate([dst, N + (pad & 127)])
    src2 = jnp.stack([src_p, src_p + M])

    x_pad = jnp.pad(x, ((0, M - N), (0, 0)))

    deg = _deg_call()(dst_p)
    dinv_col = lax.rsqrt(deg[0] + deg[1] + 1.0)[:, None]

    y1 = _mm_scale(x_pad, W1, dinv_col)
    s1 = _segsum_call()(y1.reshape(2 * M, HD), src2, dst_p)
    y2 = _mid(s1, y1, dinv_col, b1[None, :], W2)
    s2 = _segsum_call()(y2.reshape(2 * M, HD), src2, dst_p)
    return _fin(s2, y2, dinv_col, b2[None, :])


# speedup vs baseline: 1.1313x; 1.0157x over previous
import functools

import jax
import jax.numpy as jnp
from jax import lax
from jax.experimental import pallas as pl
from jax.experimental.pallas import tpu as pltpu
from jax.experimental.pallas import tpu_sc as plsc

N = 10000
D = 256
HD = 128
E = 160000
M = 10240
E_PAD = 163840
EPT = E_PAD // 16
K = 128
NCH = EPT // K
RPT = M // 16
KD = 128


@functools.cache
def _mesh():
    return plsc.VectorSubcoreMesh(
        core_axis_name="c", subcore_axis_name="s", num_cores=2, num_subcores=16
    )


def _deg_body(dst_hbm, deg_hbm, deg_sh, ones_v, idx_v, zero_v, semi0, semi1):
    c = lax.axis_index("c")
    s = lax.axis_index("s")
    for k in range(KD // 16):
        ones_v[pl.ds(k * 16, 16)] = jnp.ones((16,), jnp.float32)
    for k in range(RPT // 16):
        zero_v[pl.ds(k * 16, 16)] = jnp.zeros((16,), jnp.float32)
    pltpu.sync_copy(zero_v, deg_sh.at[pl.ds(s * RPT, RPT)])
    plsc.subcore_barrier()

    base = (s * 2 + c) * (E_PAD // 32)
    nld = E_PAD // 32 // (4 * KD)
    pltpu.async_copy(dst_hbm.at[pl.ds(base, 4 * KD)], idx_v.at[0], semi0)
    pltpu.async_copy(dst_hbm.at[pl.ds(base + 4 * KD, 4 * KD)],
                     idx_v.at[1], semi1)

    def body(jj, carry):
        for b in (0, 1):
            ld = jj * 2 + b
            sem = semi0 if b == 0 else semi1
            pltpu.make_async_copy(
                dst_hbm.at[pl.ds(base, 4 * KD)], idx_v.at[b], sem
            ).wait()
            for q in range(4):
                pltpu.sync_copy(
                    ones_v, deg_sh.at[idx_v.at[b, pl.ds(q * KD, KD)]], add=True
                )

            @pl.when(ld + 2 < nld)
            def _():
                pltpu.async_copy(
                    dst_hbm.at[pl.ds(base + (ld + 2) * 4 * KD, 4 * KD)],
                    idx_v.at[b], sem,
                )
        return carry

    lax.fori_loop(0, nld // 2, body, 0)
    plsc.subcore_barrier()
    pltpu.sync_copy(
        deg_sh.at[pl.ds(s * RPT, RPT)],
        deg_hbm.at[c, pl.ds(s * RPT, RPT)],
    )


@functools.cache
def _deg_call():
    return pl.kernel(
        _deg_body,
        out_type=jax.ShapeDtypeStruct((2, M), jnp.float32),
        mesh=_mesh(),
        scratch_types=[
            pltpu.VMEM_SHARED((M,), jnp.float32),
            pltpu.VMEM((KD,), jnp.float32),
            pltpu.VMEM((2, 4 * KD), jnp.int32),
            pltpu.VMEM((RPT,), jnp.float32),
            pltpu.SemaphoreType.DMA,
            pltpu.SemaphoreType.DMA,
        ],
        compiler_params=pltpu.CompilerParams(use_tc_tiling_on_sc=False),
    )


def _segsum_body(y_hbm, src2_hbm, dst_hbm, out_hbm,
                 acc_sh, src_in, dstbuf, rowbuf,
                 sem0, sem1, semd0, semd1, sems, sca, scb):
    c = lax.axis_index("c")
    s = lax.axis_index("s")
    base = s * EPT

    pltpu.async_copy(src2_hbm.at[c, pl.ds(base, EPT)], src_in, sems)

    def zbody(i, carry):
        for k in range(HD // 16):
            rowbuf[0, i, pl.ds(k * 16, 16)] = jnp.zeros((16,), jnp.float32)
        return carry

    lax.fori_loop(0, K, zbody, 0)
    r0 = s * RPT
    for t in range(RPT // K):
        pltpu.sync_copy(rowbuf.at[0], acc_sh.at[pl.ds(r0 + t * K, K)])
    pltpu.make_async_copy(src2_hbm.at[c, pl.ds(base, EPT)], src_in, sems).wait()
    plsc.subcore_barrier()

    pltpu.async_copy(y_hbm.at[src_in.at[pl.ds(0, K)]], rowbuf.at[0], sem0)
    pltpu.async_copy(dst_hbm.at[pl.ds(base, K)], dstbuf.at[0], semd0)
    pltpu.async_copy(y_hbm.at[src_in.at[pl.ds(K, K)]], rowbuf.at[1], sem1)
    pltpu.async_copy(dst_hbm.at[pl.ds(base + K, K)], dstbuf.at[1], semd1)

    def mbody(jj, carry):
        for b in (0, 1):
            ch = jj * 2 + b
            sem = sem0 if b == 0 else sem1
            semd = semd0 if b == 0 else semd1
            pltpu.make_async_copy(
                y_hbm.at[src_in.at[pl.ds(0, K)]], rowbuf.at[b], sem
            ).wait()
            pltpu.make_async_copy(
                dst_hbm.at[pl.ds(base, K)], dstbuf.at[b], semd
            ).wait()
            pltpu.async_copy(
                rowbuf.at[b, pl.ds(0, K // 2)],
                acc_sh.at[dstbuf.at[b, pl.ds(0, K // 2)]], sca, add=True,
            )
            pltpu.async_copy(
                rowbuf.at[b, pl.ds(K // 2, K // 2)],
                acc_sh.at[dstbuf.at[b, pl.ds(K // 2, K // 2)]], scb, add=True,
            )
            pltpu.make_async_copy(
                rowbuf.at[b, pl.ds(0, K // 2)],
                acc_sh.at[dstbuf.at[b, pl.ds(0, K // 2)]], sca,
            ).wait()
            pltpu.make_async_copy(
                rowbuf.at[b, pl.ds(K // 2, K // 2)],
                acc_sh.at[dstbuf.at[b, pl.ds(K // 2, K // 2)]], scb,
            ).wait()

            @pl.when(ch + 2 < NCH)
            def _():
                pltpu.async_copy(
                    y_hbm.at[src_in.at[pl.ds((ch + 2) * K, K)]],
                    rowbuf.at[b], sem,
                )
                pltpu.async_copy(
                    dst_hbm.at[pl.ds(base + (ch + 2) * K, K)],
                    dstbuf.at[b], semd,
                )
        return carry

    lax.fori_loop(0, NCH // 2, mbody, 0)
    plsc.subcore_barrier()

    pltpu.sync_copy(acc_sh.at[pl.ds(r0, RPT)],
                    out_hbm.at[c, pl.ds(r0, RPT)])


@functools.cache
def _segsum_call():
    return pl.kernel(
        _segsum_body,
        out_type=jax.ShapeDtypeStruct((2, M, HD), jnp.float32),
        mesh=_mesh(),
        scratch_types=[
            pltpu.VMEM_SHARED((M, HD), jnp.float32),
            pltpu.VMEM((EPT,), jnp.int32),
            pltpu.VMEM((2, K), jnp.int32),
            pltpu.VMEM((2, K, HD), jnp.float32),
        ] + [pltpu.SemaphoreType.DMA] * 7,
        compiler_params=pltpu.CompilerParams(use_tc_tiling_on_sc=False),
    )


def _mm_scale_k(x_ref, w_ref, ds_ref, y_ref):
    xw = jnp.dot(x_ref[...], w_ref[...], preferred_element_type=jnp.float32)
    xw = xw * lax.rsqrt(ds_ref[...])
    y_ref[0] = xw[:, :HD]
    y_ref[1] = xw[:, HD:]


def _mid_k(s_ref, y_ref, ds_ref, b_ref, w_ref, o_ref):
    dinv = lax.rsqrt(ds_ref[...])
    h0 = jnp.maximum(
        dinv * (s_ref[0] + y_ref[0]) + b_ref[:, :HD], 0.0
    )
    h1 = jnp.maximum(
        dinv * (s_ref[1] + y_ref[1]) + b_ref[:, HD:], 0.0
    )
    o = (
        jnp.dot(h0, w_ref[:HD, :], preferred_element_type=jnp.float32)
        + jnp.dot(h1, w_ref[HD:, :], preferred_element_type=jnp.float32)
    ) * dinv
    o_ref[0] = o[:, :HD]
    o_ref[1] = o[:, HD:]


def _fin_k(s_ref, y_ref, ds_ref, b_ref, o_ref):
    dinv = lax.rsqrt(ds_ref[...])
    o_ref[:, :HD] = jnp.maximum(
        dinv * (s_ref[0] + y_ref[0]) + b_ref[:, :HD], 0.0
    )
    o_ref[:, HD:] = jnp.maximum(
        dinv * (s_ref[1] + y_ref[1]) + b_ref[:, HD:], 0.0
    )


def _mm_scale(x, w, ds_col, bm=1000):
    g = N // bm
    return pl.pallas_call(
        _mm_scale_k,
        grid=(g,),
        in_specs=[
            pl.BlockSpec((bm, D), lambda i: (i, 0)),
            pl.BlockSpec((D, D), lambda i: (0, 0)),
            pl.BlockSpec((bm, 1), lambda i: (i, 0)),
        ],
        out_specs=pl.BlockSpec((2, bm, HD), lambda i: (0, i, 0)),
        out_shape=jax.ShapeDtypeStruct((2, M, HD), jnp.float32),
    )(x, w, ds_col)


def _mid(s1, y1, ds_col, b1, w2, bm=1024):
    g = M // bm
    return pl.pallas_call(
        _mid_k,
        grid=(g,),
        in_specs=[
            pl.BlockSpec((2, bm, HD), lambda i: (0, i, 0)),
            pl.BlockSpec((2, bm, HD), lambda i: (0, i, 0)),
            pl.BlockSpec((bm, 1), lambda i: (i, 0)),
            pl.BlockSpec((1, D), lambda i: (0, 0)),
            pl.BlockSpec((D, D), lambda i: (0, 0)),
        ],
        out_specs=pl.BlockSpec((2, bm, HD), lambda i: (0, i, 0)),
        out_shape=jax.ShapeDtypeStruct((2, M, HD), jnp.float32),
    )(s1, y1, ds_col, b1, w2)


def _fin(s2, y2, ds_col, b2, bm=2000):
    g = N // bm
    return pl.pallas_call(
        _fin_k,
        grid=(g,),
        in_specs=[
            pl.BlockSpec((2, bm, HD), lambda i: (0, i, 0)),
            pl.BlockSpec((2, bm, HD), lambda i: (0, i, 0)),
            pl.BlockSpec((bm, 1), lambda i: (i, 0)),
            pl.BlockSpec((1, D), lambda i: (0, 0)),
        ],
        out_specs=pl.BlockSpec((bm, D), lambda i: (i, 0)),
        out_shape=jax.ShapeDtypeStruct((N, D), jnp.float32),
    )(s2, y2, ds_col, b2)


@jax.jit
def kernel(x, edge_index, W1, b1, W2, b2):
    src = edge_index[0]
    dst = edge_index[1]
    pad = jnp.arange(E_PAD - E, dtype=jnp.int32)
    src_p = jnp.concatenate([src, (pad * 131) % N])
    dst_p = jnp.concatenate([dst, N + (pad & 127)])
    src2 = jnp.stack([src_p, src_p + M])

    deg = _deg_call()(dst_p)
    ds_col = (deg[0] + deg[1] + 1.0)[:, None]

    y1 = _mm_scale(x, W1, ds_col)
    s1 = _segsum_call()(y1.reshape(2 * M, HD), src2, dst_p)
    y2 = _mid(s1, y1, ds_col, b1[None, :], W2)
    s2 = _segsum_call()(y2.reshape(2 * M, HD), src2, dst_p)
    return _fin(s2, y2, ds_col, b2[None, :])


# speedup vs baseline: 1.1459x; 1.0129x over previous
import functools

import jax
import jax.numpy as jnp
from jax import lax
from jax.experimental import pallas as pl
from jax.experimental.pallas import tpu as pltpu
from jax.experimental.pallas import tpu_sc as plsc

N = 10000
D = 256
HD = 128
E = 160000
M = 10240
E_PAD = 163840
EPT = E_PAD // 16
K = 128
NCH = EPT // K
RPT = M // 16
KD = 128


@functools.cache
def _mesh():
    return plsc.VectorSubcoreMesh(
        core_axis_name="c", subcore_axis_name="s", num_cores=2, num_subcores=16
    )


def _deg_body(dst_hbm, deg_hbm, deg_sh, ones_v, idx_v, zero_v, semi0, semi1):
    c = lax.axis_index("c")
    s = lax.axis_index("s")
    for k in range(KD // 16):
        ones_v[pl.ds(k * 16, 16)] = jnp.ones((16,), jnp.float32)
    for k in range(RPT // 16):
        zero_v[pl.ds(k * 16, 16)] = jnp.zeros((16,), jnp.float32)
    pltpu.sync_copy(zero_v, deg_sh.at[pl.ds(s * RPT, RPT)])
    plsc.subcore_barrier()

    base = (s * 2 + c) * (E_PAD // 32)
    nld = E_PAD // 32 // (4 * KD)
    pltpu.async_copy(dst_hbm.at[pl.ds(base, 4 * KD)], idx_v.at[0], semi0)
    pltpu.async_copy(dst_hbm.at[pl.ds(base + 4 * KD, 4 * KD)],
                     idx_v.at[1], semi1)

    def body(jj, carry):
        for b in (0, 1):
            ld = jj * 2 + b
            sem = semi0 if b == 0 else semi1
            pltpu.make_async_copy(
                dst_hbm.at[pl.ds(base, 4 * KD)], idx_v.at[b], sem
            ).wait()
            for q in range(4):
                pltpu.sync_copy(
                    ones_v, deg_sh.at[idx_v.at[b, pl.ds(q * KD, KD)]], add=True
                )

            @pl.when(ld + 2 < nld)
            def _():
                pltpu.async_copy(
                    dst_hbm.at[pl.ds(base + (ld + 2) * 4 * KD, 4 * KD)],
                    idx_v.at[b], sem,
                )
        return carry

    lax.fori_loop(0, nld // 2, body, 0)
    plsc.subcore_barrier()
    pltpu.sync_copy(
        deg_sh.at[pl.ds(s * RPT, RPT)],
        deg_hbm.at[c, pl.ds(s * RPT, RPT)],
    )


@functools.cache
def _deg_call():
    return pl.kernel(
        _deg_body,
        out_type=jax.ShapeDtypeStruct((2, M), jnp.float32),
        mesh=_mesh(),
        scratch_types=[
            pltpu.VMEM_SHARED((M,), jnp.float32),
            pltpu.VMEM((KD,), jnp.float32),
            pltpu.VMEM((2, 4 * KD), jnp.int32),
            pltpu.VMEM((RPT,), jnp.float32),
            pltpu.SemaphoreType.DMA,
            pltpu.SemaphoreType.DMA,
        ],
        compiler_params=pltpu.CompilerParams(use_tc_tiling_on_sc=False),
    )


def _segsum_body(y_hbm, src2_hbm, dst_hbm, out_hbm,
                 acc_sh, src_in, dstbuf, rowbuf,
                 sem0, sem1, semd0, semd1, sems, sca, scb):
    c = lax.axis_index("c")
    s = lax.axis_index("s")
    base = s * EPT

    pltpu.async_copy(src2_hbm.at[c, pl.ds(base, EPT)], src_in, sems)

    def zbody(i, carry):
        for k in range(HD // 16):
            rowbuf[0, i, pl.ds(k * 16, 16)] = jnp.zeros((16,), jnp.float32)
        return carry

    lax.fori_loop(0, K, zbody, 0)
    r0 = s * RPT
    for t in range(RPT // K):
        pltpu.sync_copy(rowbuf.at[0], acc_sh.at[pl.ds(r0 + t * K, K)])
    pltpu.make_async_copy(src2_hbm.at[c, pl.ds(base, EPT)], src_in, sems).wait()
    plsc.subcore_barrier()

    pltpu.async_copy(y_hbm.at[src_in.at[pl.ds(0, K)]], rowbuf.at[0], sem0)
    pltpu.async_copy(dst_hbm.at[pl.ds(base, K)], dstbuf.at[0], semd0)
    pltpu.async_copy(y_hbm.at[src_in.at[pl.ds(K, K)]], rowbuf.at[1], sem1)
    pltpu.async_copy(dst_hbm.at[pl.ds(base + K, K)], dstbuf.at[1], semd1)

    def mbody(jj, carry):
        for b in (0, 1):
            ch = jj * 2 + b
            sem = sem0 if b == 0 else sem1
            semd = semd0 if b == 0 else semd1
            pltpu.make_async_copy(
                y_hbm.at[src_in.at[pl.ds(0, K)]], rowbuf.at[b], sem
            ).wait()
            pltpu.make_async_copy(
                dst_hbm.at[pl.ds(base, K)], dstbuf.at[b], semd
            ).wait()
            pltpu.async_copy(
                rowbuf.at[b, pl.ds(0, K // 2)],
                acc_sh.at[dstbuf.at[b, pl.ds(0, K // 2)]], sca, add=True,
            )
            pltpu.async_copy(
                rowbuf.at[b, pl.ds(K // 2, K // 2)],
                acc_sh.at[dstbuf.at[b, pl.ds(K // 2, K // 2)]], scb, add=True,
            )
            pltpu.make_async_copy(
                rowbuf.at[b, pl.ds(0, K // 2)],
                acc_sh.at[dstbuf.at[b, pl.ds(0, K // 2)]], sca,
            ).wait()
            pltpu.make_async_copy(
                rowbuf.at[b, pl.ds(K // 2, K // 2)],
                acc_sh.at[dstbuf.at[b, pl.ds(K // 2, K // 2)]], scb,
            ).wait()

            @pl.when(ch + 2 < NCH)
            def _():
                pltpu.async_copy(
                    y_hbm.at[src_in.at[pl.ds((ch + 2) * K, K)]],
                    rowbuf.at[b], sem,
                )
                pltpu.async_copy(
                    dst_hbm.at[pl.ds(base + (ch + 2) * K, K)],
                    dstbuf.at[b], semd,
                )
        return carry

    lax.fori_loop(0, NCH // 2, mbody, 0)
    plsc.subcore_barrier()

    pltpu.sync_copy(acc_sh.at[pl.ds(r0, RPT)],
                    out_hbm.at[c, pl.ds(r0, RPT)])


@functools.cache
def _segsum_call():
    return pl.kernel(
        _segsum_body,
        out_type=jax.ShapeDtypeStruct((2, M, HD), jnp.float32),
        mesh=_mesh(),
        scratch_types=[
            pltpu.VMEM_SHARED((M, HD), jnp.float32),
            pltpu.VMEM((EPT,), jnp.int32),
            pltpu.VMEM((2, K), jnp.int32),
            pltpu.VMEM((2, K, HD), jnp.float32),
        ] + [pltpu.SemaphoreType.DMA] * 7,
        compiler_params=pltpu.CompilerParams(use_tc_tiling_on_sc=False),
    )


def _mm_scale_k(x_ref, w_ref, ds_ref, y_ref):
    xw = jnp.dot(x_ref[...], w_ref[...], preferred_element_type=jnp.float32)
    xw = xw * lax.rsqrt(ds_ref[...])
    y_ref[0] = xw[:, :HD]
    y_ref[1] = xw[:, HD:]


def _mid_k(s_ref, y_ref, ds_ref, b_ref, w_ref, o_ref):
    dinv = lax.rsqrt(ds_ref[...])
    h0 = jnp.maximum(
        dinv * (s_ref[0] + y_ref[0]) + b_ref[:, :HD], 0.0
    )
    h1 = jnp.maximum(
        dinv * (s_ref[1] + y_ref[1]) + b_ref[:, HD:], 0.0
    )
    o = (
        jnp.dot(h0, w_ref[:HD, :], preferred_element_type=jnp.float32)
        + jnp.dot(h1, w_ref[HD:, :], preferred_element_type=jnp.float32)
    ) * dinv
    o_ref[0] = o[:, :HD]
    o_ref[1] = o[:, HD:]


def _fin_k(s_ref, y_ref, ds_ref, b_ref, o_ref):
    dinv = lax.rsqrt(ds_ref[...])
    o_ref[:, :HD] = jnp.maximum(
        dinv * (s_ref[0] + y_ref[0]) + b_ref[:, :HD], 0.0
    )
    o_ref[:, HD:] = jnp.maximum(
        dinv * (s_ref[1] + y_ref[1]) + b_ref[:, HD:], 0.0
    )


def _mm_scale(x, w, ds_col, bm=2000):
    g = N // bm
    return pl.pallas_call(
        _mm_scale_k,
        grid=(g,),
        in_specs=[
            pl.BlockSpec((bm, D), lambda i: (i, 0)),
            pl.BlockSpec((D, D), lambda i: (0, 0)),
            pl.BlockSpec((bm, 1), lambda i: (i, 0)),
        ],
        out_specs=pl.BlockSpec((2, bm, HD), lambda i: (0, i, 0)),
        out_shape=jax.ShapeDtypeStruct((2, M, HD), jnp.float32),
    )(x, w, ds_col)


def _mid(s1, y1, ds_col, b1, w2, bm=2048):
    g = M // bm
    return pl.pallas_call(
        _mid_k,
        grid=(g,),
        in_specs=[
            pl.BlockSpec((2, bm, HD), lambda i: (0, i, 0)),
            pl.BlockSpec((2, bm, HD), lambda i: (0, i, 0)),
            pl.BlockSpec((bm, 1), lambda i: (i, 0)),
            pl.BlockSpec((1, D), lambda i: (0, 0)),
            pl.BlockSpec((D, D), lambda i: (0, 0)),
        ],
        out_specs=pl.BlockSpec((2, bm, HD), lambda i: (0, i, 0)),
        out_shape=jax.ShapeDtypeStruct((2, M, HD), jnp.float32),
    )(s1, y1, ds_col, b1, w2)


def _fin(s2, y2, ds_col, b2, bm=2000):
    g = N // bm
    return pl.pallas_call(
        _fin_k,
        grid=(g,),
        in_specs=[
            pl.BlockSpec((2, bm, HD), lambda i: (0, i, 0)),
            pl.BlockSpec((2, bm, HD), lambda i: (0, i, 0)),
            pl.BlockSpec((bm, 1), lambda i: (i, 0)),
            pl.BlockSpec((1, D), lambda i: (0, 0)),
        ],
        out_specs=pl.BlockSpec((bm, D), lambda i: (i, 0)),
        out_shape=jax.ShapeDtypeStruct((N, D), jnp.float32),
    )(s2, y2, ds_col, b2)


@jax.jit
def kernel(x, edge_index, W1, b1, W2, b2):
    src = edge_index[0]
    dst = edge_index[1]
    pad = jnp.arange(E_PAD - E, dtype=jnp.int32)
    src_p = jnp.concatenate([src, (pad * 131) % N])
    dst_p = jnp.concatenate([dst, N + (pad & 127)])
    src2 = jnp.stack([src_p, src_p + M])

    deg = _deg_call()(dst_p)
    ds_col = (deg[0] + deg[1] + 1.0)[:, None]

    y1 = _mm_scale(x, W1, ds_col)
    s1 = _segsum_call()(y1.reshape(2 * M, HD), src2, dst_p)
    y2 = _mid(s1, y1, ds_col, b1[None, :], W2)
    s2 = _segsum_call()(y2.reshape(2 * M, HD), src2, dst_p)
    return _fin(s2, y2, ds_col, b2[None, :])
